# baseline (speedup 1.0000x reference)
import functools

import jax
import jax.numpy as jnp
from jax import lax
from jax.experimental import pallas as pl
from jax.experimental.pallas import tpu as pltpu
from jax.experimental.pallas import tpu_sc as plsc

D = 3584
N_IDS = 4096
NC, NS = 2, 16
NW = NC * NS
BPW = N_IDS // NW
CH = 8
NCHUNK = BPW // CH
NBUF = 4


@functools.partial(
    pl.kernel,
    out_type=jax.ShapeDtypeStruct((N_IDS, D), jnp.float32),
    mesh=plsc.VectorSubcoreMesh(core_axis_name="c", subcore_axis_name="s"),
    scratch_types=[
        pltpu.VMEM((BPW,), jnp.int32),
        pltpu.VMEM((NBUF, CH, D), jnp.float32),
        pltpu.SemaphoreType.DMA((NBUF,)),
        pltpu.SemaphoreType.DMA((NBUF,)),
    ],
)
def _gather_call(ids_hbm, table_hbm, out_hbm, idx_v, rows_v, in_sems, out_sems):
    wid = lax.axis_index("s") * NC + lax.axis_index("c")
    base = wid * BPW
    pltpu.sync_copy(ids_hbm.at[pl.ds(base, BPW)], idx_v)

    def gather(c, buf):
        pltpu.make_async_copy(
            table_hbm.at[idx_v.at[pl.ds(c * CH, CH)]],
            rows_v.at[buf],
            in_sems.at[buf],
        ).start()

    def wait_gather(c, buf):
        pltpu.make_async_copy(
            table_hbm.at[idx_v.at[pl.ds(c * CH, CH)]],
            rows_v.at[buf],
            in_sems.at[buf],
        ).wait()

    def put(c, buf):
        pltpu.make_async_copy(
            rows_v.at[buf],
            out_hbm.at[pl.ds(base + c * CH, CH)],
            out_sems.at[buf],
        ).start()

    def wait_put(c, buf):
        pltpu.make_async_copy(
            rows_v.at[buf],
            out_hbm.at[pl.ds(base + c * CH, CH)],
            out_sems.at[buf],
        ).wait()

    for c in range(min(NBUF - 1, NCHUNK)):
        gather(c, c % NBUF)
    for c in range(NCHUNK):
        buf = c % NBUF
        g = c + NBUF - 1
        if g < NCHUNK:
            gbuf = g % NBUF
            if g - NBUF >= 0:
                wait_put(g - NBUF, gbuf)
            gather(g, gbuf)
        wait_gather(c, buf)
        put(c, buf)
    for c in range(max(0, NCHUNK - NBUF), NCHUNK):
        wait_put(c, c % NBUF)


def kernel(input_ids, table):
    ids = input_ids.reshape(-1).astype(jnp.int32)
    out = _gather_call(ids, table)
    return out.reshape(input_ids.shape + (table.shape[1],))


# speedup vs baseline: 1.0149x; 1.0149x over previous
import functools

import jax
import jax.numpy as jnp
from jax import lax
from jax.experimental import pallas as pl
from jax.experimental.pallas import tpu as pltpu
from jax.experimental.pallas import tpu_sc as plsc

B = 2
S = 2048
D = 3584
NC, NS = 2, 16
NW = NC * NS
BPW = (B * S) // NW
WPR = S // BPW
CH = 16
NCHUNK = BPW // CH


@functools.partial(
    pl.kernel,
    out_type=jax.ShapeDtypeStruct((B, S, D), jnp.float32),
    mesh=plsc.VectorSubcoreMesh(core_axis_name="c", subcore_axis_name="s"),
    scratch_types=[
        pltpu.VMEM((BPW,), jnp.int32),
        pltpu.VMEM((2, CH, D), jnp.float32),
        pltpu.SemaphoreType.DMA((2,)),
        pltpu.SemaphoreType.DMA((2,)),
    ],
)
def _gather_call(ids_hbm, table_hbm, out_hbm, idx_v, rows_v, in_sems, out_sems):
    wid = lax.axis_index("s") * NC + lax.axis_index("c")
    b = wid // WPR
    s0 = (wid % WPR) * BPW
    pltpu.sync_copy(ids_hbm.at[b, pl.ds(s0, BPW)], idx_v)

    def gather(c, buf):
        return pltpu.make_async_copy(
            table_hbm.at[idx_v.at[pl.ds(pl.multiple_of(c * CH, 8), CH)]],
            rows_v.at[buf],
            in_sems.at[buf],
        )

    def put(c, buf):
        return pltpu.make_async_copy(
            rows_v.at[buf],
            out_hbm.at[b, pl.ds(s0 + c * CH, CH)],
            out_sems.at[buf],
        )

    gather(0, 0).start()
    gather(1, 1).start()
    gather(0, 0).wait()
    put(0, 0).start()

    @pl.loop(0, (NCHUNK - 2) // 2)
    def _(g):
        c1 = 2 * g + 1
        put(c1 - 1, 0).wait()
        gather(c1 + 1, 0).start()
        gather(c1, 1).wait()
        put(c1, 1).start()
        c2 = 2 * g + 2
        put(c2 - 1, 1).wait()
        gather(c2 + 1, 1).start()
        gather(c2, 0).wait()
        put(c2, 0).start()

    cl = NCHUNK - 1
    gather(cl, 1).wait()
    put(cl, 1).start()
    put(cl - 1, 0).wait()
    put(cl, 1).wait()


def kernel(input_ids, table):
    return _gather_call(input_ids.astype(jnp.int32), table)
